# baseline (speedup 1.0000x reference)
import jax
import jax.numpy as jnp
from jax.experimental import pallas as pl

NUM_ROWS = 16384
NUM_COLS = 1000
BLOCK_ROWS = 1024


def _onehot_body(x_ref, o_ref):
    i = pl.program_id(0)
    xs = x_ref[0, pl.ds(i * BLOCK_ROWS, BLOCK_ROWS)]
    cols = jax.lax.broadcasted_iota(jnp.int32, (BLOCK_ROWS, NUM_COLS), 1)
    o_ref[...] = (cols == xs[:, None]).astype(jnp.float32)


def kernel(x):
    x2 = x.reshape(1, NUM_ROWS).astype(jnp.int32)
    out = pl.pallas_call(
        _onehot_body,
        grid=(NUM_ROWS // BLOCK_ROWS,),
        in_specs=[pl.BlockSpec((1, NUM_ROWS), lambda i: (0, 0))],
        out_specs=pl.BlockSpec((BLOCK_ROWS, NUM_COLS), lambda i: (i, 0)),
        out_shape=jax.ShapeDtypeStruct((NUM_ROWS, NUM_COLS), jnp.float32),
    )(x2)
    return out
